# Initial kernel scaffold; baseline (speedup 1.0000x reference)
import jax
import jax.numpy as jnp
from jax.experimental import pallas as pl


def kernel(input_ids, table):
    raise NotImplementedError("write your pallas kernel here")


# speedup vs baseline: 1.8614x; 1.8614x over previous
import jax
import jax.numpy as jnp
from jax import lax
from jax.experimental import pallas as pl
from jax.experimental.pallas import tpu as pltpu
from jax.experimental.pallas import tpu_sc as plsc

_NUM_CODEBOOKS = 9
_HIDDEN = 2048
_SEMANTIC = 4096
_ACOUSTIC = 2048
_N_ACOUSTIC = 8
_AUDIO_VOCAB = 20480
_STRIDE = (_AUDIO_VOCAB - _SEMANTIC - _ACOUSTIC) // (_N_ACOUSTIC - 1)
_OFFSETS = tuple(
    0 if k == 0 else _SEMANTIC + (k - 1) * _STRIDE for k in range(_NUM_CODEBOOKS)
)

_L = 16
_NC, _NS = 2, 16
_NW = _NC * _NS
_TOKENS = 4 * 4096
_TPW = _TOKENS // _NW
_TB = 16
_NB = _TPW // _TB
_COLS = _HIDDEN // _L


def _body(ids_hbm, table_hbm, out_hbm, idxv, rows, acc, dsem):
    wid = lax.axis_index("s") * _NC + lax.axis_index("c")
    base = wid * _TPW
    pltpu.sync_copy(ids_hbm.at[:, pl.ds(base, _TPW)], idxv)
    for k in range(_NUM_CODEBOOKS):
        off = _OFFSETS[k]
        if off == 0:
            continue

        def _addoff(i, carry, k=k, off=off):
            s = i * _L
            idxv[k, pl.ds(s, _L)] = idxv[k, pl.ds(s, _L)] + off
            return carry

        lax.fori_loop(0, _TPW // _L, _addoff, None)

    def _block(b, carry):
        for k in range(_NUM_CODEBOOKS):
            pltpu.async_copy(
                table_hbm.at[idxv.at[k, pl.ds(b * _TB, _TB)]], rows, dsem
            ).wait()

            def _accum(c, inner, first=(k == 0)):
                s = c * _L
                for t in range(_TB):
                    v = rows[t, pl.ds(s, _L)]
                    if first:
                        acc[t, pl.ds(s, _L)] = v
                    else:
                        plsc.addupdate(acc.at[t, pl.ds(s, _L)], v)
                return inner

            lax.fori_loop(0, _COLS, _accum, None)
        pltpu.sync_copy(acc, out_hbm.at[pl.ds(base + b * _TB, _TB)])
        return carry

    lax.fori_loop(0, _NB, _block, None)


@jax.jit
def kernel(input_ids, table):
    ids2 = input_ids.reshape(_TOKENS, _NUM_CODEBOOKS).T
    out = pl.kernel(
        _body,
        out_type=jax.ShapeDtypeStruct((_TOKENS, _HIDDEN), jnp.float32),
        mesh=plsc.VectorSubcoreMesh(core_axis_name="c", subcore_axis_name="s"),
        scratch_types=[
            pltpu.VMEM((_NUM_CODEBOOKS, _TPW), jnp.int32),
            pltpu.VMEM((_TB, _HIDDEN), jnp.float32),
            pltpu.VMEM((_TB, _HIDDEN), jnp.float32),
            pltpu.SemaphoreType.DMA,
        ],
    )(ids2, table)
    return out.reshape(input_ids.shape[0], input_ids.shape[1], _HIDDEN)


# speedup vs baseline: 3.7353x; 2.0067x over previous
import jax
import jax.numpy as jnp
from jax import lax
from jax.experimental import pallas as pl
from jax.experimental.pallas import tpu as pltpu
from jax.experimental.pallas import tpu_sc as plsc

_NUM_CODEBOOKS = 9
_HIDDEN = 2048
_SEMANTIC = 4096
_ACOUSTIC = 2048
_N_ACOUSTIC = 8
_AUDIO_VOCAB = 20480
_STRIDE = (_AUDIO_VOCAB - _SEMANTIC - _ACOUSTIC) // (_N_ACOUSTIC - 1)
_OFFSETS = tuple(
    0 if k == 0 else _SEMANTIC + (k - 1) * _STRIDE for k in range(_NUM_CODEBOOKS)
)

_L = 16
_NC, _NS = 2, 16
_NW = _NC * _NS
_TOKENS = 4 * 4096
_TPW = _TOKENS // _NW
_TB = 16
_NB = _TPW // _TB
_COLS = _HIDDEN // _L


def _body(ids_hbm, table_hbm, out_hbm, idxv, rows0, rows1, acc, dsem0, dsem1):
    wid = lax.axis_index("s") * _NC + lax.axis_index("c")
    base = wid * _TPW
    pltpu.sync_copy(ids_hbm.at[:, pl.ds(base, _TPW)], idxv)
    for k in range(_NUM_CODEBOOKS):
        off = _OFFSETS[k]
        if off == 0:
            continue

        def _addoff(i, carry, k=k, off=off):
            s = i * _L
            idxv[k, pl.ds(s, _L)] = idxv[k, pl.ds(s, _L)] + off
            return carry

        lax.fori_loop(0, _TPW // _L, _addoff, None)

    bufs = ((rows0, dsem0), (rows1, dsem1))

    def _gather(b, k, par):
        buf, sem = bufs[par]
        return pltpu.make_async_copy(
            table_hbm.at[idxv.at[k, pl.ds(b * _TB, _TB)]], buf, sem
        )

    _gather(0, 0, 0).start()
    _gather(0, 1, 1).start()

    def _pair(p, carry):
        for blk_i in range(2):
            b = 2 * p + blk_i
            for k in range(_NUM_CODEBOOKS):
                par = (blk_i + k) % 2
                buf, _ = bufs[par]
                _gather(b, k, par).wait()

                def _accum(c, buf=buf, first=(k == 0)):
                    s = c * _L
                    for t in range(_TB):
                        v = buf[t, pl.ds(s, _L)]
                        if first:
                            acc[t, pl.ds(s, _L)] = v
                        else:
                            plsc.addupdate(acc.at[t, pl.ds(s, _L)], v)

                plsc.parallel_loop(0, _COLS, 1, unroll=2)(_accum)
                if k < _NUM_CODEBOOKS - 2:
                    _gather(b, k + 2, par).start()
                else:
                    nk = k + 2 - _NUM_CODEBOOKS

                    @pl.when(b + 1 < _NB)
                    def _start_next(b=b, nk=nk, par=par):
                        _gather(b + 1, nk, par).start()

            pltpu.sync_copy(acc, out_hbm.at[pl.ds(base + b * _TB, _TB)])
        return carry

    lax.fori_loop(0, _NB // 2, _pair, None)


@jax.jit
def kernel(input_ids, table):
    ids2 = input_ids.reshape(_TOKENS, _NUM_CODEBOOKS).T
    out = pl.kernel(
        _body,
        out_type=jax.ShapeDtypeStruct((_TOKENS, _HIDDEN), jnp.float32),
        mesh=plsc.VectorSubcoreMesh(core_axis_name="c", subcore_axis_name="s"),
        scratch_types=[
            pltpu.VMEM((_NUM_CODEBOOKS, _TPW), jnp.int32),
            pltpu.VMEM((_TB, _HIDDEN), jnp.float32),
            pltpu.VMEM((_TB, _HIDDEN), jnp.float32),
            pltpu.VMEM((_TB, _HIDDEN), jnp.float32),
            pltpu.SemaphoreType.DMA,
            pltpu.SemaphoreType.DMA,
        ],
    )(ids2, table)
    return out.reshape(input_ids.shape[0], input_ids.shape[1], _HIDDEN)
